# baseline (speedup 1.0000x reference)
import functools
import jax
import jax.numpy as jnp
from jax import lax
from jax.experimental import pallas as pl
from jax.experimental.pallas import tpu as pltpu
from jax.experimental.pallas import tpu_sc as plsc

_NW = 32
_TPW = 244
_LW = _TPW * 128
_COVER = _NW * _LW
_CAP = 704
_CAPH = 352
_LWH = _LW // 2
_IBLK = 4096


def _make_gather(B, D):
    n_iblk = B // _IBLK

    mesh = plsc.VectorSubcoreMesh(core_axis_name="c", subcore_axis_name="s")

    @functools.partial(
        pl.kernel,
        mesh=mesh,
        compiler_params=pltpu.CompilerParams(
            use_tc_tiling_on_sc=True, needs_layout_passes=False),
        out_type=[
            jax.ShapeDtypeStruct((B, D), jnp.float32),
            jax.ShapeDtypeStruct((B, D), jnp.float32),
        ],
        scratch_types=[
            pltpu.VMEM((_IBLK,), jnp.int32),
            pltpu.VMEM((2 * _CAPH,), jnp.int32),
            pltpu.VMEM((2 * _CAPH,), jnp.int32),
            pltpu.VMEM((_LWH,), jnp.float32),
            pltpu.VMEM((_LWH,), jnp.float32),
            pltpu.VMEM((_CAP, D), jnp.float32),
            pltpu.SemaphoreType.DMA,
            pltpu.SemaphoreType.DMA,
            pltpu.SemaphoreType.DMA,
            pltpu.SemaphoreType.DMA,
        ],
    )
    def k(iw_hbm, cw_hbm, tt_hbm, ct_hbm, out_i_hbm, out_c_hbm,
          iblk, sel_lane, sel_pos, cb0, cb1, stag,
          sem_i, sem_c0, sem_c1, sem_o):
        nc = plsc.get_sparse_core_info().num_cores
        wid = lax.axis_index("s") * nc + lax.axis_index("c")
        lo = wid * _LW
        lo = pl.multiple_of(lo, 128)
        cbufs = (cb0, cb1)
        csems = (sem_c0, sem_c1)
        ri = lax.iota(jnp.int32, 16)

        for idx_hbm, src_hbm, out_hbm in ((iw_hbm, tt_hbm, out_i_hbm),
                                          (cw_hbm, ct_hbm, out_c_hbm)):
            def scan_pass(wlo):
                def blk(b, carry):
                    pltpu.sync_copy(idx_hbm.at[pl.ds(b * _IBLK, _IBLK)],
                                    iblk)

                    def grp(g, c2):
                        cl, gl, ch, gh = c2
                        v16 = iblk[pl.ds(g * 16, 16)]
                        lane = v16 - lo
                        pos = b * _IBLK + g * 16 + ri
                        m = (v16 >= lo) & (v16 < lo + _LW)
                        outs = []
                        for h, (cnt2, gcnt2) in ((0, (cl, gl)),
                                                 (1, (ch, gh))):
                            if h == 0:
                                mh = m & (lane < _LWH)
                            else:
                                mh = m & (lane >= _LWH)
                            incl = plsc.cumsum(mh.astype(jnp.int32))
                            rank = gcnt2 + incl - 1
                            am = mh & (rank >= wlo) & (rank < wlo + _CAPH)
                            plsc.store_compressed(
                                sel_lane.at[pl.ds(h * _CAPH + cnt2, 16)],
                                lane - h * _LWH, mask=am)
                            plsc.store_compressed(
                                sel_pos.at[pl.ds(h * _CAPH + cnt2, 16)],
                                pos, mask=am)
                            pc_all = plsc.all_reduce_population_count(mh)[0]
                            pc_app = plsc.all_reduce_population_count(am)[0]
                            outs.append((cnt2 + pc_app, gcnt2 + pc_all))
                        return (outs[0][0], outs[0][1],
                                outs[1][0], outs[1][1])

                    return lax.fori_loop(0, _IBLK // 16, grp, c2_init(carry))

                def c2_init(c):
                    return c

                z = jnp.int32(0)
                carry = (z, z, z, z)
                for b in range(n_iblk):
                    carry = blk(b, carry)
                return carry

            def fire_chunk(c, h, p):
                off = lo + h * _LWH
                off = pl.multiple_of(off, 128)
                pltpu.async_copy(src_hbm.at[c, pl.ds(off, _LWH)],
                                 cbufs[p], csems[p])

            def wait_chunk(p):
                pltpu.make_async_copy(src_hbm.at[0, pl.ds(0, _LWH)],
                                      cbufs[p], csems[p]).wait()

            def gather_features(cl, ch):
                ngs = (lax.shift_right_logical(cl + 15, 4),
                       lax.shift_right_logical(ch + 15, 4))
                cnts = (cl, ch)

                def gather_half(c, h, p):
                    cvec = ri * 0 + c

                    def grp(g, carry):
                        sl16 = sel_lane[pl.ds(h * _CAPH + g * 16, 16)]
                        msk = (g * 16 + ri) < cnts[h]
                        vals = plsc.load_gather(cbufs[p], [sl16], mask=msk)
                        plsc.store_scatter(
                            stag, [h * _CAPH + g * 16 + ri, cvec],
                            vals, mask=msk)
                        return carry

                    lax.fori_loop(0, ngs[h], grp, None)

                fire_chunk(0, 0, 0)
                fire_chunk(0, 1, 1)

                def feat(c, carry):
                    wait_chunk(0)
                    gather_half(c, 0, 0)
                    fire_chunk(c + 1, 0, 0)
                    wait_chunk(1)
                    gather_half(c, 1, 1)
                    fire_chunk(c + 1, 1, 1)
                    return carry

                lax.fori_loop(0, D - 1, feat, None)
                wait_chunk(0)
                gather_half(D - 1, 0, 0)
                wait_chunk(1)
                gather_half(D - 1, 1, 1)

            def write_out(cl, ch):
                for h, cnt in ((0, cl), (1, ch)):
                    ng = lax.shift_right_logical(cnt + 15, 4)

                    def grp(g, carry):
                        p16 = sel_pos[pl.ds(h * _CAPH + g * 16, 16)]
                        for j in range(16):
                            @pl.when(g * 16 + j < cnt)
                            def _():
                                pltpu.async_copy(
                                    stag.at[h * _CAPH + g * 16 + j],
                                    out_hbm.at[p16[j]], sem_o)
                        return carry

                    lax.fori_loop(0, ng, grp, None)

                    def drain(g, carry):
                        for j in range(16):
                            @pl.when(g * 16 + j < cnt)
                            def _():
                                pltpu.make_async_copy(
                                    stag.at[0], out_hbm.at[0],
                                    sem_o).wait()
                        return carry

                    lax.fori_loop(0, ng, drain, None)

            def pass_body(carry):
                wlo, _gt = carry
                cl, gl, ch, gh = scan_pass(wlo)
                gather_features(cl, ch)
                write_out(cl, ch)
                return (wlo + _CAPH, jnp.maximum(gl, gh))

            def pass_cond(carry):
                wlo, gtotal = carry
                return wlo < gtotal

            lax.while_loop(pass_cond, pass_body,
                           (jnp.int32(0), jnp.int32(1)))

    return k


def kernel(input_word, context_word, target_table, context_table):
    V, D = target_table.shape
    B = input_word.shape[0]
    iw = input_word.astype(jnp.int32)
    cw = context_word.astype(jnp.int32)
    gather = _make_gather(B, D)
    out_i, out_c = gather(
        iw, cw,
        jnp.swapaxes(target_table, 0, 1),
        jnp.swapaxes(context_table, 0, 1),
    )
    tail_t = target_table[_COVER:]
    tail_c = context_table[_COVER:]
    fi = jnp.take(tail_t, jnp.clip(iw - _COVER, 0, V - _COVER - 1), axis=0)
    fc = jnp.take(tail_c, jnp.clip(cw - _COVER, 0, V - _COVER - 1), axis=0)
    out_i = jnp.where((iw >= _COVER)[:, None], fi, out_i)
    out_c = jnp.where((cw >= _COVER)[:, None], fc, out_c)
    return (out_i, out_c)


# speedup vs baseline: 1.0330x; 1.0330x over previous
import functools
import jax
import jax.numpy as jnp
from jax import lax
from jax.experimental import pallas as pl
from jax.experimental.pallas import tpu as pltpu
from jax.experimental.pallas import tpu_sc as plsc

_NW = 32
_TPW = 244
_LW = _TPW * 128
_COVER = _NW * _LW
_CAP = 704
_LWH = _LW // 2
_IBLK = 4096


def _make_gather(B, D):
    n_iblk = B // _IBLK

    mesh = plsc.VectorSubcoreMesh(core_axis_name="c", subcore_axis_name="s")

    @functools.partial(
        pl.kernel,
        mesh=mesh,
        compiler_params=pltpu.CompilerParams(
            use_tc_tiling_on_sc=True, needs_layout_passes=False),
        out_type=[
            jax.ShapeDtypeStruct((B, D), jnp.float32),
            jax.ShapeDtypeStruct((B, D), jnp.float32),
        ],
        scratch_types=[
            pltpu.VMEM((_IBLK,), jnp.int32),
            pltpu.VMEM((_CAP,), jnp.int32),
            pltpu.VMEM((_CAP,), jnp.int32),
            pltpu.VMEM((_LWH,), jnp.float32),
            pltpu.VMEM((_LWH,), jnp.float32),
            pltpu.VMEM((_CAP, D), jnp.float32),
            pltpu.SemaphoreType.DMA,
            pltpu.SemaphoreType.DMA,
            pltpu.SemaphoreType.DMA,
            pltpu.SemaphoreType.DMA,
        ],
    )
    def k(iw_hbm, cw_hbm, tt_hbm, ct_hbm, out_i_hbm, out_c_hbm,
          iblk, sel_lane, sel_pos, cb0, cb1, stag,
          sem_i, sem_c0, sem_c1, sem_o):
        nc = plsc.get_sparse_core_info().num_cores
        wid = lax.axis_index("s") * nc + lax.axis_index("c")
        lo = wid * _LW
        lo = pl.multiple_of(lo, 128)
        cbufs = (cb0, cb1)
        csems = (sem_c0, sem_c1)
        ri = lax.iota(jnp.int32, 16)

        for idx_hbm, src_hbm, out_hbm in ((iw_hbm, tt_hbm, out_i_hbm),
                                          (cw_hbm, ct_hbm, out_c_hbm)):
            def scan_pass(wlo):
                def blk(b, carry):
                    cnt, gcnt = carry
                    pltpu.sync_copy(idx_hbm.at[pl.ds(b * _IBLK, _IBLK)],
                                    iblk)

                    def grp(g, c2):
                        cnt2, gcnt2 = c2
                        v16 = iblk[pl.ds(g * 16, 16)]
                        m = (v16 >= lo) & (v16 < lo + _LW)
                        mi = m.astype(jnp.int32)
                        incl = plsc.cumsum(mi)
                        rank = gcnt2 + incl - 1
                        am = m & (rank >= wlo) & (rank < wlo + _CAP)
                        plsc.store_compressed(
                            sel_lane.at[pl.ds(cnt2, 16)], v16 - lo, mask=am)
                        plsc.store_compressed(
                            sel_pos.at[pl.ds(cnt2, 16)],
                            b * _IBLK + g * 16 + ri, mask=am)
                        pc_all = plsc.all_reduce_population_count(m)[0]
                        pc_app = plsc.all_reduce_population_count(am)[0]
                        return (cnt2 + pc_app, gcnt2 + pc_all)

                    return lax.fori_loop(0, _IBLK // 16, grp, (cnt, gcnt),
                                         unroll=4)

                cnt = jnp.int32(0)
                gcnt = jnp.int32(0)
                for b in range(n_iblk):
                    cnt, gcnt = blk(b, (cnt, gcnt))
                return cnt, gcnt

            def fire_chunk(c, h, p):
                off = lo + h * _LWH
                off = pl.multiple_of(off, 128)
                pltpu.async_copy(src_hbm.at[c, pl.ds(off, _LWH)],
                                 cbufs[p], csems[p])

            def wait_chunk(p):
                pltpu.make_async_copy(src_hbm.at[0, pl.ds(0, _LWH)],
                                      cbufs[p], csems[p]).wait()

            def gather_features(lcnt):
                ng = lax.shift_right_logical(lcnt + 63, 6)

                def gather_half(c, h, p):
                    cvec = ri * 0 + c

                    def grp(g, carry):
                        for u in range(4):
                            e0 = g * 64 + u * 16
                            sl16 = sel_lane[pl.ds(e0, 16)] - h * _LWH
                            msk = ((e0 + ri) < lcnt) & (sl16 >= 0) \
                                & (sl16 < _LWH)
                            vals = plsc.load_gather(cbufs[p], [sl16],
                                                    mask=msk)
                            plsc.store_scatter(stag, [e0 + ri, cvec],
                                               vals, mask=msk)
                        return carry

                    lax.fori_loop(0, ng, grp, None)

                fire_chunk(0, 0, 0)
                fire_chunk(0, 1, 1)

                def feat(c, carry):
                    wait_chunk(0)
                    gather_half(c, 0, 0)
                    fire_chunk(c + 1, 0, 0)
                    wait_chunk(1)
                    gather_half(c, 1, 1)
                    fire_chunk(c + 1, 1, 1)
                    return carry

                lax.fori_loop(0, D - 1, feat, None)
                wait_chunk(0)
                gather_half(D - 1, 0, 0)
                wait_chunk(1)
                gather_half(D - 1, 1, 1)

            def write_out(lcnt):
                ng = lax.shift_right_logical(lcnt + 15, 4)

                def grp(g, carry):
                    p16 = sel_pos[pl.ds(g * 16, 16)]
                    for j in range(16):
                        @pl.when(g * 16 + j < lcnt)
                        def _():
                            pltpu.async_copy(stag.at[g * 16 + j],
                                             out_hbm.at[p16[j]], sem_o)
                    return carry

                lax.fori_loop(0, ng, grp, None)

                def drain(g, carry):
                    for j in range(16):
                        @pl.when(g * 16 + j < lcnt)
                        def _():
                            pltpu.make_async_copy(
                                stag.at[0], out_hbm.at[0], sem_o).wait()
                    return carry

                lax.fori_loop(0, ng, drain, None)

            def pass_body(carry):
                wlo, _gt = carry
                lcnt, gtotal = scan_pass(wlo)
                gather_features(lcnt)
                write_out(lcnt)
                return (wlo + _CAP, gtotal)

            def pass_cond(carry):
                wlo, gtotal = carry
                return wlo < gtotal

            lax.while_loop(pass_cond, pass_body,
                           (jnp.int32(0), jnp.int32(1)))

    return k


def kernel(input_word, context_word, target_table, context_table):
    V, D = target_table.shape
    B = input_word.shape[0]
    iw = input_word.astype(jnp.int32)
    cw = context_word.astype(jnp.int32)
    gather = _make_gather(B, D)
    out_i, out_c = gather(
        iw, cw,
        jnp.swapaxes(target_table, 0, 1),
        jnp.swapaxes(context_table, 0, 1),
    )
    tail_t = target_table[_COVER:]
    tail_c = context_table[_COVER:]
    fi = jnp.take(tail_t, jnp.clip(iw - _COVER, 0, V - _COVER - 1), axis=0)
    fc = jnp.take(tail_c, jnp.clip(cw - _COVER, 0, V - _COVER - 1), axis=0)
    out_i = jnp.where((iw >= _COVER)[:, None], fi, out_i)
    out_c = jnp.where((cw >= _COVER)[:, None], fc, out_c)
    return (out_i, out_c)
